# Initial kernel scaffold; baseline (speedup 1.0000x reference)
import jax
import jax.numpy as jnp
from jax.experimental import pallas as pl


def kernel(hidden_layers, slot_embs, domains, binary_golds):
    raise NotImplementedError("write your pallas kernel here")


# speedup vs baseline: 2.9367x; 2.9367x over previous
import jax
import jax.numpy as jnp
from jax.experimental import pallas as pl
from jax.experimental.pallas import tpu as pltpu

_BSZ, _SEQ, _DIM = 16, 2048, 256
_NDOM, _NSLOT = 8, 16
_RB = 512
_NRB = _SEQ // _RB
_SR = 16
_SC = _SEQ // _SR


def _body(dom_ref, h_ref, emb_ref, lab_ref, out_ref, scores_v, seg_v):
    b = pl.program_id(0)
    rb = pl.program_id(1)

    @pl.when(rb == 0)
    def _():
        d = dom_ref[b]
        e = emb_ref[d]
        h = h_ref[0]
        scores_v[...] = jax.lax.dot_general(
            h, e, (((1,), (1,)), ((), ())),
            preferred_element_type=jnp.float32)

        lab = lab_ref[0, 0].reshape(_SR, _SC)
        is_b = jnp.where(lab == 1, 1.0, 0.0)
        r_iota = jax.lax.broadcasted_iota(jnp.int32, (_SC, _SC), 0)
        c_iota = jax.lax.broadcasted_iota(jnp.int32, (_SC, _SC), 1)
        ut = jnp.where(r_iota <= c_iota, 1.0, 0.0)
        p = jax.lax.dot_general(
            is_b, ut, (((1,), (0,)), ((), ())),
            preferred_element_type=jnp.float32)
        t = p[:, _SC - 1:_SC]
        rr = jax.lax.broadcasted_iota(jnp.int32, (_SR, _SR), 0)
        cc = jax.lax.broadcasted_iota(jnp.int32, (_SR, _SR), 1)
        lt_strict = jnp.where(cc < rr, 1.0, 0.0)
        r_off = jax.lax.dot_general(
            lt_strict, t, (((1,), (0,)), ((), ())),
            preferred_element_type=jnp.float32)
        seg = p + r_off
        kept = (lab == 1) | (lab == 2)
        seg_v[...] = jnp.where(kept, seg, 0.0)

    acc = jnp.zeros((_RB, _NSLOT), jnp.float32)
    row0 = rb * _RB + 1
    m_iota = jax.lax.broadcasted_iota(
        jnp.int32, (_RB, _SC), 0).astype(jnp.float32)
    for r in range(_SR):
        segr = seg_v[r, :][None, :]
        a = jnp.where(segr == (m_iota + row0), 1.0, 0.0)
        acc = acc + jax.lax.dot_general(
            a, scores_v[pl.ds(r * _SC, _SC), :], (((1,), (0,)), ((), ())),
            preferred_element_type=jnp.float32)
    out_ref[0] = acc


def _make():
    grid_spec = pltpu.PrefetchScalarGridSpec(
        num_scalar_prefetch=1,
        grid=(_BSZ, _NRB),
        in_specs=[
            pl.BlockSpec((1, _SEQ, _DIM), lambda b, rb, dom: (b, 0, 0)),
            pl.BlockSpec((_NDOM, _NSLOT, _DIM), lambda b, rb, dom: (0, 0, 0)),
            pl.BlockSpec((1, 1, _SEQ), lambda b, rb, dom: (b, 0, 0)),
        ],
        out_specs=pl.BlockSpec((1, _RB, _NSLOT), lambda b, rb, dom: (b, rb, 0)),
        scratch_shapes=[
            pltpu.VMEM((_SEQ, _NSLOT), jnp.float32),
            pltpu.VMEM((_SR, _SC), jnp.float32),
        ],
    )
    return pl.pallas_call(
        _body,
        grid_spec=grid_spec,
        out_shape=jax.ShapeDtypeStruct((_BSZ, _SEQ, _NSLOT), jnp.float32),
    )


def kernel(hidden_layers, slot_embs, domains, binary_golds):
    lab3 = binary_golds.astype(jnp.int32).reshape(_BSZ, 1, _SEQ)
    return _make()(domains.astype(jnp.int32), hidden_layers, slot_embs, lab3)
